# baseline (speedup 1.0000x reference)
import functools

import jax
import jax.numpy as jnp
from jax import lax
from jax.experimental import pallas as pl
from jax.experimental.pallas import tpu as pltpu
from jax.experimental.pallas import tpu_sc as plsc

_NUM_CLASSES = 100000
_D = 64
_B = 16384
_LAMBDA_C = 1.0

_L = 16
_NC = 2
_NS = 16
_NW = _NC * _NS
_BPW = _B // _NW
_ICH = 128
_NCH = _BPW // _ICH


def _make_sc_kernel():
    mesh = plsc.VectorSubcoreMesh(core_axis_name="c", subcore_axis_name="s")

    @functools.partial(
        pl.kernel,
        mesh=mesh,
        compiler_params=pltpu.CompilerParams(use_tc_tiling_on_sc=False),
        out_type=jax.ShapeDtypeStruct((_NW, _L), jnp.float32),
        scratch_types=[
            pltpu.VMEM((_NCH, _ICH), jnp.int32),
            pltpu.VMEM((_BPW, _D), jnp.float32),
            pltpu.VMEM((_BPW, _D), jnp.float32),
            pltpu.VMEM((_L,), jnp.float32),
            pltpu.SemaphoreType.DMA,
        ],
    )
    def sc_kernel(centers_hbm, y_hbm, hid_hbm, out_hbm,
                  idx_v, rows_v, hid_v, acc_v, sem):
        wid = lax.axis_index("s") * _NC + lax.axis_index("c")

        pltpu.sync_copy(y_hbm.at[wid], idx_v)
        copies = []
        for j in range(_NCH):
            copies.append(pltpu.async_copy(
                centers_hbm.at[idx_v.at[j]],
                rows_v.at[pl.ds(j * _ICH, _ICH)],
                sem))
        copies.append(pltpu.async_copy(hid_hbm.at[wid], hid_v, sem))
        for c in copies:
            c.wait()

        zero = jnp.zeros((_L,), jnp.float32)

        def body(i, accs):
            a0, a1, a2, a3 = accs
            d0 = hid_v[i, pl.ds(0, _L)] - rows_v[i, pl.ds(0, _L)]
            d1 = hid_v[i, pl.ds(_L, _L)] - rows_v[i, pl.ds(_L, _L)]
            d2 = hid_v[i, pl.ds(2 * _L, _L)] - rows_v[i, pl.ds(2 * _L, _L)]
            d3 = hid_v[i, pl.ds(3 * _L, _L)] - rows_v[i, pl.ds(3 * _L, _L)]
            return (a0 + d0 * d0, a1 + d1 * d1, a2 + d2 * d2, a3 + d3 * d3)

        a0, a1, a2, a3 = lax.fori_loop(0, _BPW, body, (zero, zero, zero, zero))
        acc_v[...] = (a0 + a1) + (a2 + a3)
        pltpu.sync_copy(acc_v, out_hbm.at[wid])

    return sc_kernel


_sc_kernel = _make_sc_kernel()


def kernel(y, hidden, centers):
    y32 = y.astype(jnp.int32).reshape(_NW, _NCH, _ICH)
    hid = hidden.reshape(_NW, _BPW, _D)
    partials = _sc_kernel(centers, y32, hid)
    return (_LAMBDA_C / 2.0 / _B) * jnp.sqrt(jnp.sum(partials))


# speedup vs baseline: 1.3464x; 1.3464x over previous
import functools

import jax
import jax.numpy as jnp
from jax import lax
from jax.experimental import pallas as pl
from jax.experimental.pallas import tpu as pltpu
from jax.experimental.pallas import tpu_sc as plsc

_NUM_CLASSES = 100000
_D = 64
_B = 16384
_LAMBDA_C = 1.0

_L = 16
_NC = 2
_NS = 16
_NW = _NC * _NS
_BPW = _B // _NW
_CH = 128
_NCHUNK = _BPW // _CH


def _make_sc_kernel():
    mesh = plsc.VectorSubcoreMesh(core_axis_name="c", subcore_axis_name="s")

    @functools.partial(
        pl.kernel,
        mesh=mesh,
        out_type=jax.ShapeDtypeStruct((_NW, _L), jnp.float32),
        scratch_types=[
            pltpu.VMEM((_BPW,), jnp.int32),
            pltpu.VMEM((_CH, _D), jnp.float32),
            pltpu.VMEM((_CH, _D), jnp.float32),
            pltpu.VMEM((_CH, _D), jnp.float32),
            pltpu.VMEM((_CH, _D), jnp.float32),
            pltpu.VMEM((_L,), jnp.float32),
            pltpu.SemaphoreType.DMA,
            pltpu.SemaphoreType.DMA,
            pltpu.SemaphoreType.DMA,
            pltpu.SemaphoreType.DMA,
        ],
    )
    def sc_kernel(centers_hbm, y_hbm, hid_hbm, out_hbm,
                  idx_v, rows0, rows1, hid0, hid1, acc_v,
                  gsem0, gsem1, hsem0, hsem1):
        rows_bufs = (rows0, rows1)
        hid_bufs = (hid0, hid1)
        gsems = (gsem0, gsem1)
        hsems = (hsem0, hsem1)

        wid = lax.axis_index("s") * _NC + lax.axis_index("c")
        base = wid * _BPW

        pltpu.sync_copy(y_hbm.at[pl.ds(base, _BPW)], idx_v)

        hcopies = [None, None]

        def issue_chunk(c):
            b = c % 2
            hcopies[b] = pltpu.async_copy(
                hid_hbm.at[pl.ds(base + c * _CH, _CH)], hid_bufs[b],
                hsems[b])

            def issue(g, carry):
                ivec = idx_v[pl.ds(c * _CH + g * _L, _L)]
                for j in range(_L):
                    pltpu.async_copy(centers_hbm.at[ivec[j]],
                                     rows_bufs[b].at[g * _L + j], gsems[b])
                return carry

            lax.fori_loop(0, _CH // _L, issue, 0)

        zero = jnp.zeros((_L,), jnp.float32)
        accs = (zero, zero, zero, zero)

        issue_chunk(0)
        for c in range(_NCHUNK):
            if c + 1 < _NCHUNK:
                issue_chunk(c + 1)
            b = c % 2
            rv, hv = rows_bufs[b], hid_bufs[b]
            pltpu.make_async_copy(
                centers_hbm.at[pl.ds(0, _CH)], rv, gsems[b]).wait()
            hcopies[b].wait()

            def body(i, accs, rv=rv, hv=hv):
                a0, a1, a2, a3 = accs
                d0 = hv[i, pl.ds(0, _L)] - rv[i, pl.ds(0, _L)]
                d1 = hv[i, pl.ds(_L, _L)] - rv[i, pl.ds(_L, _L)]
                d2 = hv[i, pl.ds(2 * _L, _L)] - rv[i, pl.ds(2 * _L, _L)]
                d3 = hv[i, pl.ds(3 * _L, _L)] - rv[i, pl.ds(3 * _L, _L)]
                return (a0 + d0 * d0, a1 + d1 * d1,
                        a2 + d2 * d2, a3 + d3 * d3)

            accs = lax.fori_loop(0, _CH, body, accs)

        a0, a1, a2, a3 = accs
        acc_v[...] = (a0 + a1) + (a2 + a3)
        pltpu.sync_copy(acc_v, out_hbm.at[wid])

    return sc_kernel


_sc_kernel = _make_sc_kernel()


def kernel(y, hidden, centers):
    partials = _sc_kernel(centers, y.astype(jnp.int32), hidden)
    return (_LAMBDA_C / 2.0 / _B) * jnp.sqrt(jnp.sum(partials))


# speedup vs baseline: 2.1913x; 1.6276x over previous
import functools

import jax
import jax.numpy as jnp
from jax import lax
from jax.experimental import pallas as pl
from jax.experimental.pallas import tpu as pltpu
from jax.experimental.pallas import tpu_sc as plsc

_NUM_CLASSES = 100000
_D = 64
_B = 16384
_LAMBDA_C = 1.0

_L = 16
_NC = 2
_NS = 16
_NW = _NC * _NS
_DPW = _D // _NW
_BH = _B // 2


def _make_sc_kernel():
    mesh = plsc.VectorSubcoreMesh(core_axis_name="c", subcore_axis_name="s")

    @functools.partial(
        pl.kernel,
        mesh=mesh,
        compiler_params=pltpu.CompilerParams(needs_layout_passes=False),
        out_type=jax.ShapeDtypeStruct((_NW, _L), jnp.float32),
        scratch_types=[
            pltpu.VMEM((_NUM_CLASSES,), jnp.float32),
            pltpu.VMEM((_BH,), jnp.int32),
            pltpu.VMEM((_BH,), jnp.float32),
            pltpu.VMEM((_L,), jnp.float32),
            pltpu.SemaphoreType.DMA,
        ],
    )
    def sc_kernel(ct_hbm, y_hbm, ht_hbm, out_hbm,
                  crow, yidx, hrow, acc_v, sem):
        wid = lax.axis_index("s") * _NC + lax.axis_index("c")

        zero = jnp.zeros((_L,), jnp.float32)
        accs = (zero, zero, zero, zero)

        for k in range(_DPW):
            d = wid * _DPW + k
            pltpu.sync_copy(ct_hbm.at[d], crow)
            for h in range(2):
                cy = pltpu.async_copy(
                    y_hbm.at[pl.ds(h * _BH, _BH)], yidx, sem)
                ch = pltpu.async_copy(
                    ht_hbm.at[d, pl.ds(h * _BH, _BH)], hrow, sem)
                cy.wait()
                ch.wait()

                def body(g, accs):
                    a0, a1, a2, a3 = accs
                    o = g * (4 * _L)
                    i0 = yidx[pl.ds(o, _L)]
                    i1 = yidx[pl.ds(o + _L, _L)]
                    i2 = yidx[pl.ds(o + 2 * _L, _L)]
                    i3 = yidx[pl.ds(o + 3 * _L, _L)]
                    d0 = hrow[pl.ds(o, _L)] - plsc.load_gather(crow, [i0])
                    d1 = hrow[pl.ds(o + _L, _L)] - plsc.load_gather(crow, [i1])
                    d2 = hrow[pl.ds(o + 2 * _L, _L)] - plsc.load_gather(
                        crow, [i2])
                    d3 = hrow[pl.ds(o + 3 * _L, _L)] - plsc.load_gather(
                        crow, [i3])
                    return (a0 + d0 * d0, a1 + d1 * d1,
                            a2 + d2 * d2, a3 + d3 * d3)

                accs = lax.fori_loop(0, _BH // (4 * _L), body, accs)

        a0, a1, a2, a3 = accs
        acc_v[...] = (a0 + a1) + (a2 + a3)
        pltpu.sync_copy(acc_v, out_hbm.at[wid])

    return sc_kernel


_sc_kernel = _make_sc_kernel()


def kernel(y, hidden, centers):
    ct = jnp.transpose(centers)
    ht = jnp.transpose(hidden)
    partials = _sc_kernel(ct, y.astype(jnp.int32), ht)
    return (_LAMBDA_C / 2.0 / _B) * jnp.sqrt(jnp.sum(partials))
